# baseline (speedup 1.0000x reference)
import functools

import jax
import jax.numpy as jnp
from jax import lax
from jax.experimental import pallas as pl
from jax.experimental.pallas import tpu as pltpu
from jax.experimental.pallas import tpu_sc as plsc

NC = 2
NS = 16
L = 16
NW = NC * NS
SUPER = 25
CAP = 16
IC = 128


def _sc_scan_body(recv_hbm, idx_out, cnt_out, recv_v, idx_v, cnt_v, rot_v,
                  slot_ref):
    E = recv_hbm.shape[0]
    chunk = E // NW
    nsuper = chunk // (L * SUPER)
    wid = lax.axis_index("s") * NC + lax.axis_index("c")
    base = wid * chunk

    pltpu.sync_copy(recv_hbm.at[pl.ds(base, chunk)], recv_v)
    cnt_v[...] = jnp.zeros((L,), jnp.int32)
    slot_ref[0] = 0

    def lane_min(m):
        for k in (8, 4, 2, 1):
            rot_v[pl.ds(0, L)] = m
            rot_v[pl.ds(L, L)] = m
            m = jnp.minimum(m, rot_v[pl.ds(L - k, L)])
        return m[0]

    def scan_group(off, vvec):
        for l in range(L):
            @pl.when(vvec[l] == 0)
            def _record():
                slot = slot_ref[0]

                @pl.when(slot < IC)
                def _store():
                    idx_v[pl.ds(slot * L, L)] = jnp.full(
                        (L,), base + off + l, jnp.int32)

                slot_ref[0] = slot + 1
                cnt_v[...] = cnt_v[...] + 1

    def super_body(sg, carry):
        off0 = sg * (L * SUPER)
        vs = [recv_v[pl.ds(off0 + u * L, L)] for u in range(SUPER)]
        while len(vs) > 1:
            folded = [jnp.minimum(a, b) for a, b in zip(vs[::2], vs[1::2])]
            if len(vs) % 2:
                folded.append(vs[-1])
            vs = folded

        @pl.when(lane_min(vs[0]) == 0)
        def _slow():
            def inner(u, c2):
                off = off0 + u * L
                g = recv_v[pl.ds(off, L)]

                @pl.when(lane_min(g) == 0)
                def _scan():
                    scan_group(off, g)
                return c2
            lax.fori_loop(0, SUPER, inner, 0)
        return carry

    lax.fori_loop(0, nsuper, super_body, 0)
    pltpu.sync_copy(idx_v, idx_out.at[wid])
    pltpu.sync_copy(cnt_v, cnt_out.at[wid])


def _bfr(a):
    return a.astype(jnp.bfloat16).astype(jnp.float32)


def _head_kernel(idx_s, cnt_s, idxall_s, attr_any, cnt2d_ref,
                 x0_ref, Wn_ref, bn_ref,
                 We_ref, be_ref, Wm0_ref, bm0_ref, Wm1_ref, bm1_ref,
                 Wu0_ref, bu0_ref, Wu1_ref, bu1_ref, Wc1_ref, bc1_ref,
                 Wc2_ref, bc2_ref, out_ref, rows_v, rowtmp_v, ovf_v, sem):
    dot = functools.partial(jax.lax.dot_general,
                            dimension_numbers=(((1,), (0,)), ((), ())),
                            preferred_element_type=jnp.float32)

    def split3(a):
        hi = _bfr(a)
        lo = _bfr(a - hi)
        l2 = _bfr(a - hi - lo)
        return hi, lo, l2

    def dot_x(a, w):
        hi, lo, l2 = split3(a)
        return dot(hi, w) + dot(lo, w) + dot(l2, w)

    def dot_xx(a, wparts):
        out = None
        for ap in split3(a):
            for wp in wparts:
                t = dot(ap, wp)
                out = t if out is None else out + t
        return out

    rows_v[...] = jnp.zeros((NW * CAP, L), jnp.float32)
    ovf_v[...] = jnp.zeros((8, L), jnp.float32)

    def issue_w(w, n):
        cw = jnp.minimum(cnt_s[w * CAP], CAP)

        def issue_j(j, nn):
            s = w * CAP + j
            pltpu.make_async_copy(attr_any.at[idx_s[s]], rows_v.at[s],
                                  sem).start()
            return nn + 1
        return lax.fori_loop(0, cw, issue_j, n)

    n_issued = lax.fori_loop(0, NW, issue_w, 0)

    def drain(i, c2):
        pltpu.make_async_copy(attr_any.at[0], rows_v.at[0], sem).wait()
        return c2
    lax.fori_loop(0, n_issued, drain, 0)

    def ovf_w(w, c2):
        cw = cnt_s[w * CAP]

        @pl.when(cw > CAP)
        def _deep():
            def deep_j(j, c3):
                pltpu.sync_copy(attr_any.at[idxall_s[w * IC + j]], rowtmp_v)
                ovf_v[0, :] = ovf_v[0, :] + rowtmp_v[...]
                return c3
            lax.fori_loop(CAP, jnp.minimum(cw, IC), deep_j, 0)
        return c2
    lax.fori_loop(0, NW, ovf_w, 0)

    cnt_vec = cnt2d_ref[...].astype(jnp.float32)
    slot = jax.lax.broadcasted_iota(jnp.int32, (NW * CAP, 1), 0) % CAP
    valid = slot.astype(jnp.float32) < cnt_vec
    c = (jnp.sum(cnt_vec) / CAP).reshape(1, 1)

    We_r = _bfr(We_ref[...])
    be = be_ref[...]

    edges_rows = dot(_bfr(rows_v[...]), We_r) + be
    edges_rb = _bfr(edges_rows)
    S2 = jnp.sum(jnp.where(valid, edges_rb, 0.0), axis=0, keepdims=True)

    s_ovf = ovf_v[0, :].reshape(1, L)
    c_ovf = (jnp.sum(jnp.maximum(cnt_vec - CAP, 0.0)) / CAP).reshape(1, 1)
    S2 = S2 + _bfr(dot_x(s_ovf, We_r) + c_ovf * be)

    agg1 = dot_x(S2, _bfr(Wm0_ref[...])) + c * bm0_ref[...]
    agg2 = dot_x(S2, _bfr(Wm1_ref[...])) + c * bm1_ref[...]
    n0 = dot(_bfr(x0_ref[...]), _bfr(Wn_ref[...])) + bn_ref[...]
    n1 = jnp.maximum(dot(_bfr(n0 + agg1), _bfr(Wu0_ref[...])) + bu0_ref[...],
                     0.0)
    n2 = jnp.maximum(dot(_bfr(n1 + agg2), _bfr(Wu1_ref[...])) + bu1_ref[...],
                     0.0)
    h = jnp.maximum(dot_xx(n2, split3(Wc1_ref[...])) + bc1_ref[...], 0.0)
    out_ref[...] = dot_xx(h, split3(Wc2_ref[...])) + bc2_ref[...]


def kernel(x, edge_attr, receivers, Wn, bn, We, be, Wm0, bm0, Wm1, bm1,
           Wu0, bu0, Wu1, bu1, Wc1, bc1, Wc2, bc2):
    mesh = plsc.VectorSubcoreMesh(core_axis_name="c", subcore_axis_name="s",
                                  num_cores=NC, num_subcores=NS)
    E = receivers.shape[0]
    idx_full, cnt = pl.kernel(
        _sc_scan_body,
        out_type=[jax.ShapeDtypeStruct((NW, IC * L), jnp.int32),
                  jax.ShapeDtypeStruct((NW, L), jnp.int32)],
        mesh=mesh,
        scratch_types=[
            pltpu.VMEM((E // NW,), jnp.int32),
            pltpu.VMEM((IC * L,), jnp.int32),
            pltpu.VMEM((L,), jnp.int32),
            pltpu.VMEM((2 * L,), jnp.int32),
            pltpu.SMEM((1,), jnp.int32),
        ],
    )(receivers)

    idx_all = idx_full[:, ::L].reshape(NW * IC)
    idx16 = idx_full[:, :CAP * L:L].reshape(NW * CAP)
    cntfl = cnt.reshape(NW * L)
    cnt2d = cnt.reshape(NW * L, 1)

    out = pl.pallas_call(
        _head_kernel,
        in_specs=[
            pl.BlockSpec(memory_space=pltpu.SMEM),
            pl.BlockSpec(memory_space=pltpu.SMEM),
            pl.BlockSpec(memory_space=pltpu.SMEM),
            pl.BlockSpec(memory_space=pltpu.HBM),
        ] + [pl.BlockSpec(memory_space=pltpu.VMEM)] * 18,
        out_specs=pl.BlockSpec(memory_space=pltpu.VMEM),
        out_shape=jax.ShapeDtypeStruct((1, 1), jnp.float32),
        scratch_shapes=[
            pltpu.VMEM((NW * CAP, L), jnp.float32),
            pltpu.VMEM((L,), jnp.float32),
            pltpu.VMEM((8, L), jnp.float32),
            pltpu.SemaphoreType.DMA,
        ],
    )(idx16, cntfl, idx_all, edge_attr, cnt2d, x[0:1], Wn,
      bn.reshape(1, 64),
      We, be.reshape(1, 64),
      Wm0, bm0.reshape(1, 64), Wm1, bm1.reshape(1, 64),
      Wu0, bu0.reshape(1, 64), Wu1, bu1.reshape(1, 64),
      Wc1, bc1.reshape(1, 32), Wc2, bc2.reshape(1, 1))
    return out[0, 0]
